# Initial kernel scaffold; baseline (speedup 1.0000x reference)
import jax
import jax.numpy as jnp
from jax.experimental import pallas as pl


def kernel(poi_seq, category_seq, hour_seq, poi_table, cat_table, hour_table, W, b):
    raise NotImplementedError("write your pallas kernel here")


# speedup vs baseline: 2.6907x; 2.6907x over previous
import functools

import jax
import jax.numpy as jnp
from jax import lax
from jax.experimental import pallas as pl
from jax.experimental.pallas import tpu as pltpu
from jax.experimental.pallas import tpu_sc as plsc

NC, NS = 2, 16
NW = NC * NS
CHUNK = 128


def _sc_gather_sum(poi_table, cat_table, poi_idx, cat_idx):
    N = poi_idx.shape[0]
    D = poi_table.shape[1]
    n_per_w = N // NW
    n_chunks = n_per_w // CHUNK
    mesh = plsc.VectorSubcoreMesh(core_axis_name="c", subcore_axis_name="s")

    @functools.partial(
        pl.kernel,
        out_type=jax.ShapeDtypeStruct((N, D), jnp.float32),
        mesh=mesh,
        compiler_params=pltpu.CompilerParams(use_tc_tiling_on_sc=False),
        scratch_types=[
            pltpu.VMEM((n_per_w,), jnp.int32),
            pltpu.VMEM((n_per_w,), jnp.int32),
            pltpu.VMEM((CHUNK, D), jnp.float32),
            pltpu.SemaphoreType.DMA,
            pltpu.SemaphoreType.DMA,
        ],
    )
    def k(poi_t, cat_t, pidx_h, cidx_h, out_h, pidx_v, cidx_v, buf, sem_c, sem_p):
        wid = lax.axis_index("s") * NC + lax.axis_index("c")
        base = wid * n_per_w
        pltpu.sync_copy(pidx_h.at[pl.ds(base, n_per_w)], pidx_v)
        pltpu.sync_copy(cidx_h.at[pl.ds(base, n_per_w)], cidx_v)

        def body(j, carry):
            off = j * CHUNK
            pltpu.async_copy(
                cat_t.at[cidx_v.at[pl.ds(off, CHUNK)]], buf, sem_c).wait()
            pltpu.async_copy(
                poi_t.at[pidx_v.at[pl.ds(off, CHUNK)]], buf, sem_p,
                add=True).wait()
            pltpu.sync_copy(buf, out_h.at[pl.ds(base + off, CHUNK)])
            return carry

        lax.fori_loop(0, n_chunks, body, 0)

    return k(poi_table, cat_table, poi_idx, cat_idx)


def _tc_dense(token_emb, hour3, W, hour_pad, b, TB):
    N, D = token_emb.shape
    NB = N // TB
    H = hour_pad.shape[0]

    def body(x_ref, h_ref, w_ref, hp_ref, b_ref, o_ref):
        x = x_ref[...]
        h = h_ref[0, 0, :]
        Wfull = w_ref[...]
        hp = lax.dot_general(
            hp_ref[...], Wfull[:, D:], (((1,), (1,)), ((), ())),
            preferred_element_type=jnp.float32)
        hp = hp + b_ref[...][None, :]
        oh = (h[:, None] == lax.broadcasted_iota(jnp.int32, (TB, H), 1)
              ).astype(jnp.float32)
        y = lax.dot_general(
            x, Wfull[:, :D], (((1,), (1,)), ((), ())),
            preferred_element_type=jnp.float32)
        y = y + lax.dot_general(
            oh, hp, (((1,), (0,)), ((), ())),
            preferred_element_type=jnp.float32)
        o_ref[...] = jnp.tanh(y)

    return pl.pallas_call(
        body,
        grid=(NB,),
        in_specs=[
            pl.BlockSpec((TB, D), lambda i: (i, 0)),
            pl.BlockSpec((1, 1, TB), lambda i: (i, 0, 0)),
            pl.BlockSpec(W.shape, lambda i: (0, 0)),
            pl.BlockSpec(hour_pad.shape, lambda i: (0, 0)),
            pl.BlockSpec(b.shape, lambda i: (0,)),
        ],
        out_specs=pl.BlockSpec((TB, D), lambda i: (i, 0)),
        out_shape=jax.ShapeDtypeStruct((N, D), jnp.float32),
    )(token_emb, hour3, W, hour_pad, b)


def kernel(poi_seq, category_seq, hour_seq, poi_table, cat_table, hour_table, W, b):
    B_, L_ = poi_seq.shape
    D = poi_table.shape[1]
    N = B_ * L_

    pidx = poi_seq.reshape(N).astype(jnp.int32)
    cidx = category_seq.reshape(N).astype(jnp.int32)
    token_emb = _sc_gather_sum(poi_table, cat_table, pidx, cidx)

    TB = 1024
    hour3 = hour_seq.astype(jnp.int32).reshape(N // TB, 1, TB)
    hour_pad = jnp.pad(hour_table, ((0, 32 - hour_table.shape[0]), (0, 0)))
    out = _tc_dense(token_emb, hour3, W, hour_pad, b, TB)
    return out.reshape(B_, L_, D)


# speedup vs baseline: 2.9124x; 1.0824x over previous
import functools

import jax
import jax.numpy as jnp
from jax import lax
from jax.experimental import pallas as pl
from jax.experimental.pallas import tpu as pltpu
from jax.experimental.pallas import tpu_sc as plsc

NC, NS = 2, 16
NW = NC * NS
CHUNK = 128


def _sc_gather_sum(poi_table, cat_table, poi_idx, cat_idx):
    N = poi_idx.shape[0]
    D = poi_table.shape[1]
    n_per_w = N // NW
    n_chunks = n_per_w // CHUNK
    mesh = plsc.VectorSubcoreMesh(core_axis_name="c", subcore_axis_name="s")

    K = 8
    n_groups = n_chunks // K

    @functools.partial(
        pl.kernel,
        out_type=jax.ShapeDtypeStruct((N, D), jnp.float32),
        mesh=mesh,
        compiler_params=pltpu.CompilerParams(use_tc_tiling_on_sc=False),
        scratch_types=[
            pltpu.VMEM((n_per_w,), jnp.int32),
            pltpu.VMEM((n_per_w,), jnp.int32),
            pltpu.VMEM((K, CHUNK, D), jnp.float32),
            pltpu.SemaphoreType.DMA,
            pltpu.SemaphoreType.DMA,
            pltpu.SemaphoreType.DMA,
        ],
    )
    def k(poi_t, cat_t, pidx_h, cidx_h, out_h, pidx_v, cidx_v, bufs,
          sem_c, sem_p, sem_o):
        wid = lax.axis_index("s") * NC + lax.axis_index("c")
        base = wid * n_per_w
        pltpu.sync_copy(pidx_h.at[pl.ds(base, n_per_w)], pidx_v)
        pltpu.sync_copy(cidx_h.at[pl.ds(base, n_per_w)], cidx_v)

        def out_descs(g):
            return [
                pltpu.make_async_copy(
                    bufs.at[s],
                    out_h.at[pl.ds(base + (g * K + s) * CHUNK, CHUNK)],
                    sem_o)
                for s in range(K)
            ]

        def body(g, carry):
            @pl.when(g > 0)
            def _():
                for d in out_descs(g - 1):
                    d.wait()

            cats = [
                pltpu.async_copy(
                    cat_t.at[cidx_v.at[pl.ds((g * K + s) * CHUNK, CHUNK)]],
                    bufs.at[s], sem_c)
                for s in range(K)
            ]
            for d in cats:
                d.wait()
            pois = [
                pltpu.async_copy(
                    poi_t.at[pidx_v.at[pl.ds((g * K + s) * CHUNK, CHUNK)]],
                    bufs.at[s], sem_p, add=True)
                for s in range(K)
            ]
            for d in pois:
                d.wait()
            for d in out_descs(g):
                d.start()
            return carry

        lax.fori_loop(0, n_groups, body, 0)
        for d in out_descs(n_groups - 1):
            d.wait()

    return k(poi_table, cat_table, poi_idx, cat_idx)


def _tc_dense(token_emb, hour3, W, hour_pad, b, TB):
    N, D = token_emb.shape
    NB = N // TB
    H = hour_pad.shape[0]

    def body(x_ref, h_ref, w_ref, hp_ref, b_ref, o_ref):
        x = x_ref[...]
        h = h_ref[0, 0, :]
        Wfull = w_ref[...]
        hp = lax.dot_general(
            hp_ref[...], Wfull[:, D:], (((1,), (1,)), ((), ())),
            preferred_element_type=jnp.float32)
        hp = hp + b_ref[...][None, :]
        oh = (h[:, None] == lax.broadcasted_iota(jnp.int32, (TB, H), 1)
              ).astype(jnp.float32)
        y = lax.dot_general(
            x, Wfull[:, :D], (((1,), (1,)), ((), ())),
            preferred_element_type=jnp.float32)
        y = y + lax.dot_general(
            oh, hp, (((1,), (0,)), ((), ())),
            preferred_element_type=jnp.float32)
        o_ref[...] = jnp.tanh(y)

    return pl.pallas_call(
        body,
        grid=(NB,),
        in_specs=[
            pl.BlockSpec((TB, D), lambda i: (i, 0)),
            pl.BlockSpec((1, 1, TB), lambda i: (i, 0, 0)),
            pl.BlockSpec(W.shape, lambda i: (0, 0)),
            pl.BlockSpec(hour_pad.shape, lambda i: (0, 0)),
            pl.BlockSpec(b.shape, lambda i: (0,)),
        ],
        out_specs=pl.BlockSpec((TB, D), lambda i: (i, 0)),
        out_shape=jax.ShapeDtypeStruct((N, D), jnp.float32),
    )(token_emb, hour3, W, hour_pad, b)


def kernel(poi_seq, category_seq, hour_seq, poi_table, cat_table, hour_table, W, b):
    B_, L_ = poi_seq.shape
    D = poi_table.shape[1]
    N = B_ * L_

    pidx = poi_seq.reshape(N).astype(jnp.int32)
    cidx = category_seq.reshape(N).astype(jnp.int32)
    token_emb = _sc_gather_sum(poi_table, cat_table, pidx, cidx)

    TB = 1024
    hour3 = hour_seq.astype(jnp.int32).reshape(N // TB, 1, TB)
    hour_pad = jnp.pad(hour_table, ((0, 32 - hour_table.shape[0]), (0, 0)))
    out = _tc_dense(token_emb, hour3, W, hour_pad, b, TB)
    return out.reshape(B_, L_, D)


# speedup vs baseline: 3.5700x; 1.2258x over previous
import functools

import jax
import jax.numpy as jnp
from jax import lax
from jax.experimental import pallas as pl
from jax.experimental.pallas import tpu as pltpu
from jax.experimental.pallas import tpu_sc as plsc

NC, NS = 2, 16
NW = NC * NS
CHUNK = 128


def _sc_gather_sum(poi_table, cat_table, poi_idx, cat_idx):
    N = poi_idx.shape[0]
    D = poi_table.shape[1]
    n_per_w = N // NW
    n_chunks = n_per_w // CHUNK
    K = 8
    n_groups = n_chunks // K
    mesh = plsc.VectorSubcoreMesh(core_axis_name="c", subcore_axis_name="s")

    @functools.partial(
        pl.kernel,
        out_type=jax.ShapeDtypeStruct((N // 2, 2 * D), jnp.float32),
        mesh=mesh,
        compiler_params=pltpu.CompilerParams(use_tc_tiling_on_sc=False),
        scratch_types=[
            pltpu.VMEM((n_per_w,), jnp.int32),
            pltpu.VMEM((n_per_w,), jnp.int32),
            pltpu.VMEM((K, CHUNK, D), jnp.float32),
            pltpu.SemaphoreType.DMA,
            pltpu.SemaphoreType.DMA,
            pltpu.SemaphoreType.DMA,
        ],
    )
    def k(poi_t, cat_t, pidx_h, cidx_h, out_h, pidx_v, cidx_v, bufs,
          sem_c, sem_p, sem_o):
        wid = lax.axis_index("s") * NC + lax.axis_index("c")
        base = wid * n_per_w
        rowbase = (wid % 16) * n_per_w
        col0 = (wid // 16) * D
        pltpu.sync_copy(pidx_h.at[pl.ds(base, n_per_w)], pidx_v)
        pltpu.sync_copy(cidx_h.at[pl.ds(base, n_per_w)], cidx_v)

        def out_descs(g):
            return [
                pltpu.make_async_copy(
                    bufs.at[s],
                    out_h.at[pl.ds(rowbase + (g * K + s) * CHUNK, CHUNK),
                             pl.ds(col0, D)],
                    sem_o)
                for s in range(K)
            ]

        def body(g, carry):
            @pl.when(g > 0)
            def _():
                for d in out_descs(g - 1):
                    d.wait()

            cats = [
                pltpu.async_copy(
                    cat_t.at[cidx_v.at[pl.ds((g * K + s) * CHUNK, CHUNK)]],
                    bufs.at[s], sem_c)
                for s in range(K)
            ]
            for d in cats:
                d.wait()
            pois = [
                pltpu.async_copy(
                    poi_t.at[pidx_v.at[pl.ds((g * K + s) * CHUNK, CHUNK)]],
                    bufs.at[s], sem_p, add=True)
                for s in range(K)
            ]
            for d in pois:
                d.wait()
            for d in out_descs(g):
                d.start()
            return carry

        lax.fori_loop(0, n_groups, body, 0)
        for d in out_descs(n_groups - 1):
            d.wait()

    return k(poi_table, cat_table, poi_idx, cat_idx)


def _tc_dense(packed, hour3, W, hour_pad, b, TBp):
    M, D2 = packed.shape
    D = D2 // 2
    NBp = M // TBp
    H = hour_pad.shape[0]

    def body(x_ref, hl_ref, hr_ref, w_ref, hp_ref, b_ref, o_ref):
        x = x_ref[...]
        Wfull = w_ref[...]
        hp = lax.dot_general(
            hp_ref[...], Wfull[:, D:], (((1,), (1,)), ((), ())),
            preferred_element_type=jnp.float32)
        hp = hp + b_ref[...][None, :]
        iota = lax.broadcasted_iota(jnp.int32, (TBp, H), 1)
        for half, h_ref in ((0, hl_ref), (1, hr_ref)):
            h = h_ref[0, 0, :]
            oh = (h[:, None] == iota).astype(jnp.float32)
            y = lax.dot_general(
                x[:, half * D:(half + 1) * D], Wfull[:, :D],
                (((1,), (1,)), ((), ())),
                preferred_element_type=jnp.float32)
            y = y + lax.dot_general(
                oh, hp, (((1,), (0,)), ((), ())),
                preferred_element_type=jnp.float32)
            o_ref[half, :, :] = jnp.tanh(y)

    return pl.pallas_call(
        body,
        grid=(NBp,),
        in_specs=[
            pl.BlockSpec((TBp, D2), lambda i: (i, 0)),
            pl.BlockSpec((1, 1, TBp), lambda i: (i, 0, 0)),
            pl.BlockSpec((1, 1, TBp), lambda i: (i + NBp, 0, 0)),
            pl.BlockSpec(W.shape, lambda i: (0, 0)),
            pl.BlockSpec(hour_pad.shape, lambda i: (0, 0)),
            pl.BlockSpec(b.shape, lambda i: (0,)),
        ],
        out_specs=pl.BlockSpec((2, TBp, D), lambda i: (0, i, 0)),
        out_shape=jax.ShapeDtypeStruct((2, M, D), jnp.float32),
    )(packed, hour3, hour3, W, hour_pad, b)


def kernel(poi_seq, category_seq, hour_seq, poi_table, cat_table, hour_table, W, b):
    B_, L_ = poi_seq.shape
    D = poi_table.shape[1]
    N = B_ * L_

    pidx = poi_seq.reshape(N).astype(jnp.int32)
    cidx = category_seq.reshape(N).astype(jnp.int32)
    packed = _sc_gather_sum(poi_table, cat_table, pidx, cidx)

    TBp = 512
    hour3 = hour_seq.astype(jnp.int32).reshape(N // TBp, 1, TBp)
    hour_pad = jnp.pad(hour_table, ((0, 32 - hour_table.shape[0]), (0, 0)))
    out = _tc_dense(packed, hour3, W, hour_pad, b, TBp)
    return out.reshape(B_, L_, D)
